# baseline (speedup 1.0000x reference)
import functools

import jax
import jax.numpy as jnp
from jax import lax
from jax.experimental import pallas as pl
from jax.experimental.pallas import tpu as pltpu
from jax.experimental.pallas import tpu_sc as plsc

NUM_FRAMES = 100000
NUM_CHANNELS = 32
BATCH = 16384

_NUM_CORES = 2
_NUM_SUBCORES = 16
_LANES = 16

_OCHUNK = 2048
_N_CHUNKS = BATCH // _OCHUNK
_VECS_PER_CHUNK = _OCHUNK // _LANES

_mesh = plsc.VectorSubcoreMesh(core_axis_name="c", subcore_axis_name="s")


@functools.partial(
    pl.kernel,
    mesh=_mesh,
    out_type=jax.ShapeDtypeStruct((NUM_CHANNELS, BATCH), jnp.float32),
    scratch_types=[
        pltpu.VMEM((NUM_FRAMES,), jnp.float32),
        pltpu.VMEM((BATCH,), jnp.int32),
        pltpu.VMEM((_OCHUNK,), jnp.float32),
        pltpu.VMEM((_OCHUNK,), jnp.float32),
        pltpu.SemaphoreType.DMA,
        pltpu.SemaphoreType.DMA,
        pltpu.SemaphoreType.DMA,
    ],
    compiler_params=pltpu.CompilerParams(
        use_tc_tiling_on_sc=True, needs_layout_passes=False
    ),
)
def _gather_kernel(
    idx_hbm, emb_t_hbm, out_hbm, row_v, idx_v, ob0, ob1, sem_in, sem_o0, sem_o1
):
    c = lax.axis_index("s") * _NUM_CORES + lax.axis_index("c")
    cp_row = pltpu.async_copy(emb_t_hbm.at[c], row_v, sem_in)
    cp_idx = pltpu.async_copy(idx_hbm, idx_v, sem_in)
    cp_idx.wait()
    cp_row.wait()

    bufs = (ob0, ob1)
    out_sems = (sem_o0, sem_o1)
    pending = [None, None]
    for j in range(_N_CHUNKS):
        slot = j % 2
        buf = bufs[slot]
        if pending[slot] is not None:
            pending[slot].wait()

        @plsc.parallel_loop(0, _VECS_PER_CHUNK, unroll=16)
        def _vec(k, _j=j, _buf=buf):
            vidx = idx_v[pl.ds(_j * _OCHUNK + k * _LANES, _LANES)]
            _buf[pl.ds(k * _LANES, _LANES)] = plsc.load_gather(row_v, [vidx])

        pending[slot] = pltpu.async_copy(
            buf,
            out_hbm.at[c, pl.ds(j * _OCHUNK, _OCHUNK)],
            out_sems[slot],
        )
    pending[0].wait()
    pending[1].wait()


def kernel(idx, embedding):
    return _gather_kernel(idx, embedding.T).T


# speedup vs baseline: 1.0255x; 1.0255x over previous
import functools

import jax
import jax.numpy as jnp
from jax import lax
from jax.experimental import pallas as pl
from jax.experimental.pallas import tpu as pltpu
from jax.experimental.pallas import tpu_sc as plsc

NUM_FRAMES = 100000
NUM_CHANNELS = 32
BATCH = 16384

_NUM_CORES = 2
_NUM_SUBCORES = 16
_LANES = 16

_OCHUNK = 2048
_N_CHUNKS = BATCH // _OCHUNK
_VECS_PER_CHUNK = _OCHUNK // _LANES

_mesh = plsc.VectorSubcoreMesh(core_axis_name="c", subcore_axis_name="s")


@functools.partial(
    pl.kernel,
    mesh=_mesh,
    out_type=jax.ShapeDtypeStruct((NUM_CHANNELS, BATCH), jnp.float32),
    scratch_types=[
        pltpu.VMEM((NUM_FRAMES,), jnp.float32),
        pltpu.VMEM((BATCH,), jnp.int32),
        pltpu.VMEM((_OCHUNK,), jnp.float32),
        pltpu.VMEM((_OCHUNK,), jnp.float32),
        pltpu.SemaphoreType.DMA,
        pltpu.SemaphoreType.DMA,
        pltpu.SemaphoreType.DMA,
    ],
    compiler_params=pltpu.CompilerParams(
        use_tc_tiling_on_sc=True, needs_layout_passes=False
    ),
)
def _gather_kernel(
    idx_hbm, emb_t_hbm, out_hbm, row_v, idx_v, ob0, ob1, sem_in, sem_o0, sem_o1
):
    c = lax.axis_index("s") * _NUM_CORES + lax.axis_index("c")
    cp_row = pltpu.async_copy(emb_t_hbm.at[c], row_v, sem_in)
    cp_idx = pltpu.async_copy(idx_hbm, idx_v, sem_in)
    cp_idx.wait()
    cp_row.wait()

    bufs = (ob0, ob1)
    out_sems = (sem_o0, sem_o1)
    pending = [None, None]
    for j in range(_N_CHUNKS):
        slot = j % 2
        buf = bufs[slot]
        if pending[slot] is not None:
            pending[slot].wait()

        @plsc.parallel_loop(0, _VECS_PER_CHUNK, unroll=8)
        def _vec(k, _j=j, _buf=buf):
            vidx = idx_v[pl.ds(_j * _OCHUNK + k * _LANES, _LANES)]
            _buf[pl.ds(k * _LANES, _LANES)] = plsc.load_gather(row_v, [vidx])

        pending[slot] = pltpu.async_copy(
            buf,
            out_hbm.at[c, pl.ds(j * _OCHUNK, _OCHUNK)],
            out_sems[slot],
        )
    pending[0].wait()
    pending[1].wait()


def kernel(idx, embedding):
    return _gather_kernel(idx, embedding.T).T


# speedup vs baseline: 1.0474x; 1.0213x over previous
import functools

import jax
import jax.numpy as jnp
from jax import lax
from jax.experimental import pallas as pl
from jax.experimental.pallas import tpu as pltpu
from jax.experimental.pallas import tpu_sc as plsc

NUM_FRAMES = 100000
NUM_CHANNELS = 32
BATCH = 16384

_NUM_CORES = 2
_NUM_SUBCORES = 16
_LANES = 16

_OCHUNK = 4096
_N_CHUNKS = BATCH // _OCHUNK
_VECS_PER_CHUNK = _OCHUNK // _LANES

_mesh = plsc.VectorSubcoreMesh(core_axis_name="c", subcore_axis_name="s")


@functools.partial(
    pl.kernel,
    mesh=_mesh,
    out_type=jax.ShapeDtypeStruct((NUM_CHANNELS, BATCH), jnp.float32),
    scratch_types=[
        pltpu.VMEM((NUM_FRAMES,), jnp.float32),
        pltpu.VMEM((BATCH,), jnp.int32),
        pltpu.VMEM((_OCHUNK,), jnp.float32),
        pltpu.VMEM((_OCHUNK,), jnp.float32),
        pltpu.SemaphoreType.DMA,
        pltpu.SemaphoreType.DMA,
        pltpu.SemaphoreType.DMA,
    ],
    compiler_params=pltpu.CompilerParams(
        use_tc_tiling_on_sc=True, needs_layout_passes=False
    ),
)
def _gather_kernel(
    idx_hbm, emb_t_hbm, out_hbm, row_v, idx_v, ob0, ob1, sem_in, sem_o0, sem_o1
):
    c = lax.axis_index("s") * _NUM_CORES + lax.axis_index("c")
    cp_idx = pltpu.async_copy(idx_hbm, idx_v, sem_in)
    cp_row = pltpu.async_copy(emb_t_hbm.at[c], row_v, sem_in)
    cp_idx.wait()
    cp_row.wait()

    bufs = (ob0, ob1)
    out_sems = (sem_o0, sem_o1)
    pending = [None, None]
    for j in range(_N_CHUNKS):
        slot = j % 2
        buf = bufs[slot]
        if pending[slot] is not None:
            pending[slot].wait()

        @plsc.parallel_loop(0, _VECS_PER_CHUNK, unroll=8)
        def _vec(k, _j=j, _buf=buf):
            vidx = idx_v[pl.ds(_j * _OCHUNK + k * _LANES, _LANES)]
            _buf[pl.ds(k * _LANES, _LANES)] = plsc.load_gather(row_v, [vidx])

        pending[slot] = pltpu.async_copy(
            buf,
            out_hbm.at[c, pl.ds(j * _OCHUNK, _OCHUNK)],
            out_sems[slot],
        )
    pending[0].wait()
    pending[1].wait()


def kernel(idx, embedding):
    return _gather_kernel(idx, embedding.T).T
